# baseline (speedup 1.0000x reference)
import functools

import jax
import jax.numpy as jnp
from jax import lax
from jax.experimental import pallas as pl
from jax.experimental.pallas import tpu as pltpu
from jax.experimental.pallas import tpu_sc as plsc

N = 10000
E = 320000
H = 128
R = 200
RRELU_SLOPE = (1.0 / 8.0 + 1.0 / 3.0) / 2.0

NC = 2
NS = 16
NW = NC * NS

WROWS = 320
NPAD = NW * WROWS
CCHUNK = 8000
NCC = E // CCHUNK
EPW = E // NW
K = 80
NCH = EPW // K
G = 25
NG = NCH // G
SROWS = 632
APAD = NS * SROWS


def _sc_mesh():
    return plsc.VectorSubcoreMesh(core_axis_name="c", subcore_axis_name="s")


@functools.partial(
    pl.kernel,
    out_type=jax.ShapeDtypeStruct((NPAD * R,), jnp.float32),
    mesh=_sc_mesh(),
    compiler_params=pltpu.CompilerParams(needs_layout_passes=False),
    scratch_types=[
        pltpu.VMEM((WROWS * R,), jnp.float32),
        pltpu.VMEM((CCHUNK,), jnp.int32),
        pltpu.VMEM((CCHUNK,), jnp.int32),
        pltpu.VMEM((CCHUNK,), jnp.int32),
        pltpu.VMEM((CCHUNK,), jnp.int32),
        pltpu.SemaphoreType.DMA,
        pltpu.SemaphoreType.DMA,
    ],
)
def _sc_counts(dst_hbm, typ_hbm, zero_hbm, c_hbm, cbuf,
               dbuf0, tbuf0, dbuf1, tbuf1, sem0, sem1):
    w = lax.axis_index("c") * NS + lax.axis_index("s")
    base = w * WROWS
    pltpu.sync_copy(zero_hbm, cbuf)
    ones = jnp.full((16,), 1.0, jnp.float32)
    dbufs = (dbuf0, dbuf1)
    tbufs = (tbuf0, tbuf1)
    sems = (sem0, sem1)

    def stage(i, b):
        pltpu.async_copy(dst_hbm.at[pl.ds(i * CCHUNK, CCHUNK)],
                         dbufs[b], sems[b])
        pltpu.async_copy(typ_hbm.at[pl.ds(i * CCHUNK, CCHUNK)],
                         tbufs[b], sems[b])

    def wait_stage(i, b):
        pltpu.make_async_copy(dst_hbm.at[pl.ds(i * CCHUNK, CCHUNK)],
                              dbufs[b], sems[b]).wait()
        pltpu.make_async_copy(typ_hbm.at[pl.ds(i * CCHUNK, CCHUNK)],
                              tbufs[b], sems[b]).wait()

    def process(b):
        UNROLL = 5

        def vec(j5, _):
            for u in range(UNROLL):
                j = j5 * UNROLL + u
                d = dbufs[b][pl.ds(j * 16, 16)]
                t = tbufs[b][pl.ds(j * 16, 16)]
                local = d - base
                mask = (local >= 0) & (local < WROWS)
                lc = jnp.minimum(jnp.maximum(local, 0), WROWS - 1)
                plsc.addupdate_scatter(cbuf, [lc * R + t], ones, mask=mask)
            return 0

        lax.fori_loop(0, CCHUNK // 16 // UNROLL, vec, 0)

    stage(0, 0)

    def pair(i2, _):
        i = i2 * 2
        stage(i + 1, 1)
        wait_stage(i, 0)
        process(0)

        @pl.when(i2 < NCC // 2 - 1)
        def _():
            stage(i + 2, 0)

        wait_stage(i + 1, 1)
        process(1)
        return 0

    lax.fori_loop(0, NCC // 2, pair, 0)
    pltpu.sync_copy(cbuf, c_hbm.at[pl.ds(base * R, WROWS * R)])


@functools.partial(
    pl.kernel,
    out_type=[jax.ShapeDtypeStruct((APAD, H), jnp.float32),
              jax.ShapeDtypeStruct((APAD, H), jnp.float32)],
    mesh=_sc_mesh(),
    compiler_params=pltpu.CompilerParams(needs_layout_passes=False),
    scratch_types=[
        pltpu.VMEM_SHARED((APAD, H), jnp.float32),
        pltpu.VMEM((G * K,), jnp.int32),
        pltpu.VMEM((G * K,), jnp.int32),
        pltpu.VMEM((K, H), jnp.float32),
        pltpu.VMEM((K, H), jnp.float32),
        pltpu.SemaphoreType.DMA,
        pltpu.SemaphoreType.DMA,
        pltpu.SemaphoreType.DMA,
        pltpu.SemaphoreType.DMA,
    ],
)
def _sc_scatter(hw_hbm, src_hbm, dst_hbm, zero_hbm, out0_hbm, out1_hbm,
                agg, sbuf, dbuf, rows0, rows1, gsem0, gsem1, ssem0, ssem1):
    c = lax.axis_index("c")
    s = lax.axis_index("s")
    w = c * NS + s
    pltpu.sync_copy(zero_hbm, agg.at[pl.ds(s * SROWS, SROWS)])
    plsc.subcore_barrier()

    def gather0(i):
        pltpu.async_copy(hw_hbm.at[sbuf.at[pl.ds(i * K, K)]], rows0, gsem0)

    def gather1(i):
        pltpu.async_copy(hw_hbm.at[sbuf.at[pl.ds(i * K, K)]], rows1, gsem1)

    def wait0(i):
        pltpu.make_async_copy(hw_hbm.at[sbuf.at[pl.ds(i * K, K)]],
                              rows0, gsem0).wait()

    def wait1(i):
        pltpu.make_async_copy(hw_hbm.at[sbuf.at[pl.ds(i * K, K)]],
                              rows1, gsem1).wait()

    def scat(rows, i, ssem):
        pltpu.async_copy(rows, agg.at[dbuf.at[pl.ds(i * K, K)]], ssem,
                         add=True)

    def scat_wait(rows, i, ssem):
        pltpu.make_async_copy(rows, agg.at[dbuf.at[pl.ds(i * K, K)]],
                              ssem).wait()

    def group(g, _):
        off = w * EPW + g * (G * K)
        pltpu.sync_copy(src_hbm.at[pl.ds(off, G * K)], sbuf)
        pltpu.sync_copy(dst_hbm.at[pl.ds(off, G * K)], dbuf)
        gather0(0)
        gather1(1)

        def pair(i2, _):
            i = i2 * 2
            wait0(i)
            scat(rows0, i, ssem0)
            wait1(i + 1)
            scat(rows1, i + 1, ssem1)
            scat_wait(rows0, i, ssem0)
            gather0(i + 2)
            scat_wait(rows1, i + 1, ssem1)

            @pl.when(i + 3 < G)
            def _():
                gather1(i + 3)

            return 0

        lax.fori_loop(0, G // 2, pair, 0)
        wait0(G - 1)
        pltpu.sync_copy(rows0, agg.at[dbuf.at[pl.ds((G - 1) * K, K)]],
                        add=True)
        return 0

    lax.fori_loop(0, NG, group, 0)
    plsc.subcore_barrier()

    @pl.when(c == 0)
    def _():
        pltpu.sync_copy(agg.at[pl.ds(s * SROWS, SROWS)],
                        out0_hbm.at[pl.ds(s * SROWS, SROWS)])

    @pl.when(c == 1)
    def _():
        pltpu.sync_copy(agg.at[pl.ds(s * SROWS, SROWS)],
                        out1_hbm.at[pl.ds(s * SROWS, SROWS)])


BN = 1000


def _selfloop_and_rest(h, C, rel, wn, wl, we, norm, hw_out, rest_out):
    f32 = jnp.float32
    rW = jnp.dot(rel, wn, preferred_element_type=f32)
    hw_out[...] = jnp.dot(h, wn, preferred_element_type=f32)
    aggrel = jnp.dot(C, rW, preferred_element_type=f32)
    indeg = jnp.sum(C, axis=1, keepdims=True)
    loop_msg = jnp.where(indeg > 0,
                         jnp.dot(h, wl, preferred_element_type=f32),
                         jnp.dot(h, we, preferred_element_type=f32))
    rest_out[...] = aggrel * norm + loop_msg


def _rrelu(x):
    return jnp.where(x >= 0, x, x * RRELU_SLOPE)


def _tc_layer0_body(h_ref, c_ref, rel_ref,
                    wn_ref, wl_ref, we_ref, norm_ref, hw_out, rest_out):
    _selfloop_and_rest(h_ref[...], c_ref[...],
                       rel_ref[...], wn_ref[...],
                       wl_ref[...], we_ref[...], norm_ref[...],
                       hw_out, rest_out)


def _tc_layer1_body(pa_ref, pb_ref, restp_ref, c_ref, rel_ref,
                    wn_ref, wl_ref, we_ref, norm_ref, hw_out, rest_out):
    h = _rrelu((pa_ref[...] + pb_ref[...]) * norm_ref[...] + restp_ref[...])
    _selfloop_and_rest(h, c_ref[...],
                       rel_ref[...], wn_ref[...],
                       wl_ref[...], we_ref[...], norm_ref[...],
                       hw_out, rest_out)


def _tc_final_body(pa_ref, pb_ref, rest_ref, norm_ref, out_ref):
    out_ref[...] = _rrelu((pa_ref[...] + pb_ref[...]) * norm_ref[...]
                          + rest_ref[...])


def _row_spec(cols):
    return pl.BlockSpec((BN, cols), lambda i: (i, 0))


def _full_spec(rows, cols):
    return pl.BlockSpec((rows, cols), lambda i: (0, 0))


_GRID = N // BN


def _tc_layer0(h, C, rel, wn, wl, we, norm):
    return pl.pallas_call(
        _tc_layer0_body,
        grid=(_GRID,),
        in_specs=[_row_spec(H), _row_spec(R),
                  _full_spec(R, H), _full_spec(H, H), _full_spec(H, H),
                  _full_spec(H, H), _row_spec(1)],
        out_specs=[_row_spec(H), _row_spec(H)],
        out_shape=[jax.ShapeDtypeStruct((N, H), jnp.float32),
                   jax.ShapeDtypeStruct((N, H), jnp.float32)],
    )(h, C, rel, wn, wl, we, norm)


def _tc_layer1(pa, pb, restp, C, rel, wn, wl, we, norm):
    return pl.pallas_call(
        _tc_layer1_body,
        grid=(_GRID,),
        in_specs=[_row_spec(H), _row_spec(H), _row_spec(H),
                  _row_spec(R),
                  _full_spec(R, H), _full_spec(H, H), _full_spec(H, H),
                  _full_spec(H, H), _row_spec(1)],
        out_specs=[_row_spec(H), _row_spec(H)],
        out_shape=[jax.ShapeDtypeStruct((N, H), jnp.float32),
                   jax.ShapeDtypeStruct((N, H), jnp.float32)],
    )(pa, pb, restp, C, rel, wn, wl, we, norm)


def _tc_final(pa, pb, rest, norm):
    return pl.pallas_call(
        _tc_final_body,
        grid=(_GRID,),
        in_specs=[_row_spec(H), _row_spec(H), _row_spec(H), _row_spec(1)],
        out_specs=_row_spec(H),
        out_shape=jax.ShapeDtypeStruct((N, H), jnp.float32),
    )(pa, pb, rest, norm)


@jax.jit
def _run(node_id, edge_index, edge_type, norm, init_ent_emb, init_rel_emb,
         W_neighbor_0, loop_weight_0, evolve_loop_weight_0,
         W_neighbor_1, loop_weight_1, evolve_loop_weight_1):
    h0 = init_ent_emb
    src3 = edge_index[0]
    dst3 = edge_index[1]
    zc = jnp.zeros((WROWS * R,), jnp.float32)
    zs = jnp.zeros((SROWS, H), jnp.float32)

    C = _sc_counts(dst3, edge_type, zc).reshape(NPAD, R)

    hw0, rest0 = _tc_layer0(h0, C, init_rel_emb,
                            W_neighbor_0, loop_weight_0,
                            evolve_loop_weight_0, norm)
    pa0, pb0 = _sc_scatter(hw0, src3, dst3, zs)
    hw1, rest1 = _tc_layer1(pa0, pb0, rest0, C, init_rel_emb,
                            W_neighbor_1, loop_weight_1,
                            evolve_loop_weight_1, norm)
    pa1, pb1 = _sc_scatter(hw1, src3, dst3, zs)
    return _tc_final(pa1, pb1, rest1, norm)


def kernel(node_id, edge_index, edge_type, norm, init_ent_emb, init_rel_emb,
           W_neighbor_0, loop_weight_0, evolve_loop_weight_0,
           W_neighbor_1, loop_weight_1, evolve_loop_weight_1):
    return _run(node_id, edge_index, edge_type, norm, init_ent_emb,
                init_rel_emb, W_neighbor_0, loop_weight_0,
                evolve_loop_weight_0, W_neighbor_1, loop_weight_1,
                evolve_loop_weight_1)


# speedup vs baseline: 1.1954x; 1.1954x over previous
import functools

import jax
import jax.numpy as jnp
from jax import lax
from jax.experimental import pallas as pl
from jax.experimental.pallas import tpu as pltpu
from jax.experimental.pallas import tpu_sc as plsc

N = 10000
E = 320000
H = 128
R = 200
RRELU_SLOPE = (1.0 / 8.0 + 1.0 / 3.0) / 2.0

NC = 2
NS = 16
NW = NC * NS

WROWS = 320
NPAD = NW * WROWS
CCHUNK = 8000
NCC = E // CCHUNK
EPW = E // NW
K = 80
NCH = EPW // K
G = 25
NG = NCH // G
SROWS = 632
APAD = NS * SROWS


def _sc_mesh():
    return plsc.VectorSubcoreMesh(core_axis_name="c", subcore_axis_name="s")


@functools.partial(
    pl.kernel,
    out_type=jax.ShapeDtypeStruct((NPAD * R,), jnp.float32),
    mesh=_sc_mesh(),
    compiler_params=pltpu.CompilerParams(needs_layout_passes=False),
    scratch_types=[
        pltpu.VMEM((WROWS * R,), jnp.float32),
        pltpu.VMEM((CCHUNK,), jnp.int32),
        pltpu.VMEM((CCHUNK,), jnp.int32),
        pltpu.VMEM((CCHUNK,), jnp.int32),
        pltpu.VMEM((CCHUNK,), jnp.int32),
        pltpu.SemaphoreType.DMA,
        pltpu.SemaphoreType.DMA,
    ],
)
def _sc_counts(dst_hbm, typ_hbm, zero_hbm, c_hbm, cbuf,
               dbuf0, tbuf0, dbuf1, tbuf1, sem0, sem1):
    w = lax.axis_index("c") * NS + lax.axis_index("s")
    base = w * WROWS
    pltpu.sync_copy(zero_hbm, cbuf)
    ones = jnp.full((16,), 1.0, jnp.float32)
    dbufs = (dbuf0, dbuf1)
    tbufs = (tbuf0, tbuf1)
    sems = (sem0, sem1)

    def stage(i, b):
        pltpu.async_copy(dst_hbm.at[pl.ds(i * CCHUNK, CCHUNK)],
                         dbufs[b], sems[b])
        pltpu.async_copy(typ_hbm.at[pl.ds(i * CCHUNK, CCHUNK)],
                         tbufs[b], sems[b])

    def wait_stage(i, b):
        pltpu.make_async_copy(dst_hbm.at[pl.ds(i * CCHUNK, CCHUNK)],
                              dbufs[b], sems[b]).wait()
        pltpu.make_async_copy(typ_hbm.at[pl.ds(i * CCHUNK, CCHUNK)],
                              tbufs[b], sems[b]).wait()

    def process(b):
        UNROLL = 5

        def vec(j5, _):
            for u in range(UNROLL):
                j = j5 * UNROLL + u
                d = dbufs[b][pl.ds(j * 16, 16)]
                t = tbufs[b][pl.ds(j * 16, 16)]
                local = d - base
                mask = (local >= 0) & (local < WROWS)
                lc = jnp.minimum(jnp.maximum(local, 0), WROWS - 1)
                plsc.addupdate_scatter(cbuf, [lc * R + t], ones, mask=mask)
            return 0

        lax.fori_loop(0, CCHUNK // 16 // UNROLL, vec, 0)

    stage(0, 0)

    def pair(i2, _):
        i = i2 * 2
        stage(i + 1, 1)
        wait_stage(i, 0)
        process(0)

        @pl.when(i2 < NCC // 2 - 1)
        def _():
            stage(i + 2, 0)

        wait_stage(i + 1, 1)
        process(1)
        return 0

    lax.fori_loop(0, NCC // 2, pair, 0)
    pltpu.sync_copy(cbuf, c_hbm.at[pl.ds(base * R, WROWS * R)])


@functools.partial(
    pl.kernel,
    out_type=[jax.ShapeDtypeStruct((APAD, H), jnp.float32),
              jax.ShapeDtypeStruct((APAD, H), jnp.float32)],
    mesh=_sc_mesh(),
    compiler_params=pltpu.CompilerParams(needs_layout_passes=False),
    scratch_types=[
        pltpu.VMEM_SHARED((APAD, H), jnp.float32),
        pltpu.VMEM((G * K,), jnp.int32),
        pltpu.VMEM((G * K,), jnp.int32),
        pltpu.VMEM((K, H), jnp.float32),
        pltpu.VMEM((K, H), jnp.float32),
        pltpu.VMEM((K, H), jnp.float32),
        pltpu.SemaphoreType.DMA,
        pltpu.SemaphoreType.DMA,
        pltpu.SemaphoreType.DMA,
    ],
)
def _sc_scatter(hw_hbm, src_hbm, dst_hbm, zero_hbm, out0_hbm, out1_hbm,
                agg, sbuf, dbuf, rows0, rows1, rows2, gsem0, gsem1, gsem2):
    c = lax.axis_index("c")
    s = lax.axis_index("s")
    w = c * NS + s
    pltpu.sync_copy(zero_hbm, agg.at[pl.ds(s * SROWS, SROWS)])
    plsc.subcore_barrier()

    rowsb = (rows0, rows1, rows2)
    gsems = (gsem0, gsem1, gsem2)

    def gather(b, i):
        pltpu.async_copy(hw_hbm.at[sbuf.at[pl.ds(i * K, K)]],
                         rowsb[b], gsems[b])

    def gwait(b, i):
        pltpu.make_async_copy(hw_hbm.at[sbuf.at[pl.ds(i * K, K)]],
                              rowsb[b], gsems[b]).wait()

    def scat(b, i):
        pltpu.sync_copy(rowsb[b], agg.at[dbuf.at[pl.ds(i * K, K)]],
                        add=True)

    def group(g, _):
        off = w * EPW + g * (G * K)
        pltpu.sync_copy(src_hbm.at[pl.ds(off, G * K)], sbuf)
        pltpu.sync_copy(dst_hbm.at[pl.ds(off, G * K)], dbuf)
        gather(0, 0)
        gather(1, 1)
        gather(2, 2)

        def triple(i3, _):
            i = i3 * 3
            for b in range(3):
                gwait(b, i + b)
                scat(b, i + b)

                @pl.when(i + b + 3 < G)
                def _():
                    gather(b, i + b + 3)

            return 0

        lax.fori_loop(0, G // 3, triple, 0)
        gwait(0, G - 1)
        scat(0, G - 1)
        return 0

    lax.fori_loop(0, NG, group, 0)
    plsc.subcore_barrier()

    @pl.when(c == 0)
    def _():
        pltpu.sync_copy(agg.at[pl.ds(s * SROWS, SROWS)],
                        out0_hbm.at[pl.ds(s * SROWS, SROWS)])

    @pl.when(c == 1)
    def _():
        pltpu.sync_copy(agg.at[pl.ds(s * SROWS, SROWS)],
                        out1_hbm.at[pl.ds(s * SROWS, SROWS)])


BN = 1000


def _selfloop_and_rest(h, C, rel, wn, wl, we, norm, hw_out, rest_out):
    f32 = jnp.float32
    rW = jnp.dot(rel, wn, preferred_element_type=f32)
    hw_out[...] = jnp.dot(h, wn, preferred_element_type=f32)
    aggrel = jnp.dot(C, rW, preferred_element_type=f32)
    indeg = jnp.sum(C, axis=1, keepdims=True)
    loop_msg = jnp.where(indeg > 0,
                         jnp.dot(h, wl, preferred_element_type=f32),
                         jnp.dot(h, we, preferred_element_type=f32))
    rest_out[...] = aggrel * norm + loop_msg


def _rrelu(x):
    return jnp.where(x >= 0, x, x * RRELU_SLOPE)


def _tc_layer0_body(h_ref, c_ref, rel_ref,
                    wn_ref, wl_ref, we_ref, norm_ref, hw_out, rest_out):
    _selfloop_and_rest(h_ref[...], c_ref[...],
                       rel_ref[...], wn_ref[...],
                       wl_ref[...], we_ref[...], norm_ref[...],
                       hw_out, rest_out)


def _tc_layer1_body(pa_ref, pb_ref, restp_ref, c_ref, rel_ref,
                    wn_ref, wl_ref, we_ref, norm_ref, hw_out, rest_out):
    h = _rrelu((pa_ref[...] + pb_ref[...]) * norm_ref[...] + restp_ref[...])
    _selfloop_and_rest(h, c_ref[...],
                       rel_ref[...], wn_ref[...],
                       wl_ref[...], we_ref[...], norm_ref[...],
                       hw_out, rest_out)


def _tc_final_body(pa_ref, pb_ref, rest_ref, norm_ref, out_ref):
    out_ref[...] = _rrelu((pa_ref[...] + pb_ref[...]) * norm_ref[...]
                          + rest_ref[...])


def _row_spec(cols):
    return pl.BlockSpec((BN, cols), lambda i: (i, 0))


def _full_spec(rows, cols):
    return pl.BlockSpec((rows, cols), lambda i: (0, 0))


_GRID = N // BN


def _tc_layer0(h, C, rel, wn, wl, we, norm):
    return pl.pallas_call(
        _tc_layer0_body,
        grid=(_GRID,),
        in_specs=[_row_spec(H), _row_spec(R),
                  _full_spec(R, H), _full_spec(H, H), _full_spec(H, H),
                  _full_spec(H, H), _row_spec(1)],
        out_specs=[_row_spec(H), _row_spec(H)],
        out_shape=[jax.ShapeDtypeStruct((N, H), jnp.float32),
                   jax.ShapeDtypeStruct((N, H), jnp.float32)],
    )(h, C, rel, wn, wl, we, norm)


def _tc_layer1(pa, pb, restp, C, rel, wn, wl, we, norm):
    return pl.pallas_call(
        _tc_layer1_body,
        grid=(_GRID,),
        in_specs=[_row_spec(H), _row_spec(H), _row_spec(H),
                  _row_spec(R),
                  _full_spec(R, H), _full_spec(H, H), _full_spec(H, H),
                  _full_spec(H, H), _row_spec(1)],
        out_specs=[_row_spec(H), _row_spec(H)],
        out_shape=[jax.ShapeDtypeStruct((N, H), jnp.float32),
                   jax.ShapeDtypeStruct((N, H), jnp.float32)],
    )(pa, pb, restp, C, rel, wn, wl, we, norm)


def _tc_final(pa, pb, rest, norm):
    return pl.pallas_call(
        _tc_final_body,
        grid=(_GRID,),
        in_specs=[_row_spec(H), _row_spec(H), _row_spec(H), _row_spec(1)],
        out_specs=_row_spec(H),
        out_shape=jax.ShapeDtypeStruct((N, H), jnp.float32),
    )(pa, pb, rest, norm)


@jax.jit
def _run(node_id, edge_index, edge_type, norm, init_ent_emb, init_rel_emb,
         W_neighbor_0, loop_weight_0, evolve_loop_weight_0,
         W_neighbor_1, loop_weight_1, evolve_loop_weight_1):
    h0 = init_ent_emb
    src3 = edge_index[0]
    dst3 = edge_index[1]
    zc = jnp.zeros((WROWS * R,), jnp.float32)
    zs = jnp.zeros((SROWS, H), jnp.float32)

    C = _sc_counts(dst3, edge_type, zc).reshape(NPAD, R)

    hw0, rest0 = _tc_layer0(h0, C, init_rel_emb,
                            W_neighbor_0, loop_weight_0,
                            evolve_loop_weight_0, norm)
    pa0, pb0 = _sc_scatter(hw0, src3, dst3, zs)
    hw1, rest1 = _tc_layer1(pa0, pb0, rest0, C, init_rel_emb,
                            W_neighbor_1, loop_weight_1,
                            evolve_loop_weight_1, norm)
    pa1, pb1 = _sc_scatter(hw1, src3, dst3, zs)
    return _tc_final(pa1, pb1, rest1, norm)


def kernel(node_id, edge_index, edge_type, norm, init_ent_emb, init_rel_emb,
           W_neighbor_0, loop_weight_0, evolve_loop_weight_0,
           W_neighbor_1, loop_weight_1, evolve_loop_weight_1):
    return _run(node_id, edge_index, edge_type, norm, init_ent_emb,
                init_rel_emb, W_neighbor_0, loop_weight_0,
                evolve_loop_weight_0, W_neighbor_1, loop_weight_1,
                evolve_loop_weight_1)
